# baseline (speedup 1.0000x reference)
import jax
import jax.numpy as jnp
from jax.experimental import pallas as pl
from jax.experimental.pallas import tpu as pltpu

_GRID = 64


def _consume(xc_ref, xe_ref, o_ref):
    s = (jnp.sum(xc_ref[...], axis=0, keepdims=True)
         + jnp.sum(xe_ref[...], axis=0, keepdims=True))
    o_ref[...] = jnp.broadcast_to(s, o_ref.shape)


@jax.jit
def kernel(cell_attr, edge_index, edge_attr,
           c_w1, c_b1, c_w2, c_b2, c_w3, c_b3, c_gamma, c_beta,
           e_w1, e_b1, e_w2, e_b2, e_w3, e_b3, e_gamma, e_beta):
    n_c = cell_attr.shape[0]
    n_e = edge_attr.shape[0]
    tc = n_c // _GRID
    te = n_e // _GRID
    o = pl.pallas_call(
        _consume,
        out_shape=jax.ShapeDtypeStruct((_GRID * 8, 32), jnp.float32),
        grid=(_GRID,),
        in_specs=[pl.BlockSpec((tc, 32), lambda i: (i, 0)),
                  pl.BlockSpec((te, 32), lambda i: (i, 0))],
        out_specs=pl.BlockSpec((8, 32), lambda i: (i, 0)),
        compiler_params=pltpu.CompilerParams(
            dimension_semantics=("parallel",)),
    )(cell_attr, edge_attr)
    return {"x": o}
